# baseline (speedup 1.0000x reference)
import functools

import jax
import jax.numpy as jnp
from jax import lax
from jax.experimental import pallas as pl
from jax.experimental.pallas import tpu as pltpu
from jax.experimental.pallas import tpu_sc as plsc

N_NODES = 10000
N_EDGES = 320000
CH = 128

NC, NS, L = 2, 16, 16
NW = NC * NS
N_PAD = 10240
E_PAD = 327680
EPT = E_PAD // NW
K = 64
N_CHUNKS = EPT // K
SPT = N_PAD // NS

NBUF = 4
D = 2
ISLOTS = 8
IL = 6
BODYN = 8

N0 = 264
N1 = 320 - N0
assert N0 % BODYN == 0 and N1 % BODYN == 0
assert NS * (N0 + N1) * K == E_PAD

ROW_BLK = 1024
N_BLKS = N_PAD // ROW_BLK


@functools.cache
def _sc_kernels():
    mesh = plsc.VectorSubcoreMesh(core_axis_name="c", subcore_axis_name="s",
                                  num_cores=NC, num_subcores=NS)


    @functools.partial(
        pl.kernel,
        out_type=jax.ShapeDtypeStruct((NW, N_PAD), jnp.float32),
        mesh=mesh,
        scratch_types=[
            pltpu.VMEM((N_PAD,), jnp.float32),
            pltpu.VMEM((EPT,), jnp.int32),
        ],
        compiler_params=pltpu.CompilerParams(needs_layout_passes=False),
    )
    def deg_kernel(dst_hbm, out_hbm, hist_v, idx_v):
        cid = lax.axis_index("c")
        sid = lax.axis_index("s")
        wid = sid * NC + cid
        zeros = jnp.zeros((L,), jnp.float32)
        ones = jnp.ones((L,), jnp.float32)

        def zero_body(i, carry):
            hist_v[pl.ds(i * L, L)] = zeros
            return carry
        lax.fori_loop(0, N_PAD // L, zero_body, 0, unroll=8)

        pltpu.sync_copy(dst_hbm.at[pl.ds(wid * EPT, EPT)], idx_v)

        def hist_body(i, carry):
            idx = idx_v[pl.ds(i * L, L)]
            plsc.addupdate_scatter(hist_v, [idx], ones)
            return carry
        lax.fori_loop(0, EPT // L, hist_body, 0, unroll=8)

        pltpu.sync_copy(hist_v, out_hbm.at[wid])


    @functools.partial(
        pl.kernel,
        out_type=jax.ShapeDtypeStruct((NC, N_PAD, CH), jnp.float32),
        mesh=mesh,
        scratch_types=[
            pltpu.VMEM((ISLOTS, 2, K), jnp.int32),
            pltpu.VMEM((NBUF, K, CH), jnp.float32),
            pltpu.VMEM_SHARED((N_PAD, CH), jnp.float32),
            pltpu.SemaphoreType.DMA((ISLOTS,)),
            pltpu.SemaphoreType.DMA((NBUF,)),
            pltpu.SemaphoreType.DMA((NBUF,)),
        ],
    )
    def scatter_kernel(sd_hbm, y_hbm, out_hbm,
                       sdv, rows, acc_sh, isem, gsem, ssem):
        cid = lax.axis_index("c")
        sid = lax.axis_index("s")
        cbase = jnp.where(cid == 0, sid * N0, NS * N0 + sid * N1)
        nch = jnp.where(cid == 0, N0, N1)

        zeros = jnp.zeros((L,), jnp.float32)

        def zb(i, carry):
            rows[0, i // (CH // L), pl.ds((i % (CH // L)) * L, L)] = zeros
            return carry
        lax.fori_loop(0, K * (CH // L), zb, 0, unroll=8)

        def zcopy(j, carry):
            pltpu.sync_copy(rows.at[0],
                            acc_sh.at[pl.ds(sid * SPT + j * K, K)])
            return carry
        lax.fori_loop(0, SPT // K, zcopy, 0)

        plsc.subcore_barrier()

        def idx_copy(c, s):
            return pltpu.make_async_copy(
                sd_hbm.at[cbase + c], sdv.at[s], isem.at[s])

        def gather_copy(r, s):
            return pltpu.make_async_copy(
                y_hbm.at[sdv.at[s, 0]], rows.at[r], gsem.at[r])

        def scatter_copy(r, s):
            return pltpu.make_async_copy(
                rows.at[r], acc_sh.at[sdv.at[s, 1]], ssem.at[r])

        for c in range(IL):
            idx_copy(c, c % ISLOTS).start()
        for c in range(D):
            idx_copy(c, c % ISLOTS).wait()
            gather_copy(c % NBUF, c % ISLOTS).start()

        def step(c, j):
            r, s = j % NBUF, j % ISLOTS
            gather_copy(r, s).wait()
            scatter_copy(r, s).start(add=True)

            @pl.when(c - (NBUF - D) >= 0)
            def _():
                scatter_copy((j + D) % NBUF, (j + D) % ISLOTS).wait()

            @pl.when(c + D < nch)
            def _():
                idx_copy(c + D, (j + D) % ISLOTS).wait()
                gather_copy((j + D) % NBUF, (j + D) % ISLOTS).start()

            @pl.when(c + IL < nch)
            def _():
                idx_copy(c + IL, (j + IL) % ISLOTS).start()

        def block(t, carry):
            for j in range(BODYN):
                step(t * BODYN + j, j)
            return carry
        lax.fori_loop(0, nch // BODYN, block, 0)

        for i in range(NBUF - D):
            c = BODYN - (NBUF - D) + i
            scatter_copy(c % NBUF, c % ISLOTS).wait()

        plsc.subcore_barrier()

        pltpu.sync_copy(acc_sh.at[pl.ds(sid * SPT, SPT)],
                        out_hbm.at[cid].at[pl.ds(sid * SPT, SPT)])

    return deg_kernel, scatter_kernel


def _y_body(x_ref, w_ref, p_ref, y_ref):
    deg = 1.0 + jnp.sum(p_ref[...], axis=0)
    dinv = lax.rsqrt(deg)
    xw = jnp.dot(x_ref[...], w_ref[...], preferred_element_type=jnp.float32)
    y_ref[...] = xw * dinv[:, None]


_y_call = pl.pallas_call(
    _y_body,
    out_shape=jax.ShapeDtypeStruct((N_PAD, CH), jnp.float32),
    grid=(N_BLKS,),
    in_specs=[
        pl.BlockSpec((ROW_BLK, CH), lambda i: (i, 0)),
        pl.BlockSpec((CH, CH), lambda i: (0, 0)),
        pl.BlockSpec((NW, ROW_BLK), lambda i: (0, i)),
    ],
    out_specs=pl.BlockSpec((ROW_BLK, CH), lambda i: (i, 0)),
)


def _out_body(a_ref, y_ref, p_ref, b_ref, o_ref):
    deg = 1.0 + jnp.sum(p_ref[...], axis=0)
    dinv = lax.rsqrt(deg)
    s = a_ref[0] + a_ref[1] + y_ref[...]
    o_ref[...] = jnp.maximum(s * dinv[:, None] + b_ref[...], 0.0)


_out_call = pl.pallas_call(
    _out_body,
    out_shape=jax.ShapeDtypeStruct((N_PAD, CH), jnp.float32),
    grid=(N_BLKS,),
    in_specs=[
        pl.BlockSpec((NC, ROW_BLK, CH), lambda i: (0, i, 0)),
        pl.BlockSpec((ROW_BLK, CH), lambda i: (i, 0)),
        pl.BlockSpec((NW, ROW_BLK), lambda i: (0, i)),
        pl.BlockSpec((1, CH), lambda i: (0, 0)),
    ],
    out_specs=pl.BlockSpec((ROW_BLK, CH), lambda i: (i, 0)),
)


def kernel(x, edge_index, W, b):
    deg_kernel, scatter_kernel = _sc_kernels()
    src = edge_index[0].astype(jnp.int32)
    dst = edge_index[1].astype(jnp.int32)
    pad = jnp.full((E_PAD - N_EDGES,), N_PAD - 1, jnp.int32)
    src_p = jnp.concatenate([src, pad])
    dst_p = jnp.concatenate([dst, pad])
    x_p = jnp.pad(x, ((0, N_PAD - N_NODES), (0, 0)))

    partials = deg_kernel(dst_p)
    y = _y_call(x_p, W, partials)
    sd = jnp.stack([src_p.reshape(E_PAD // K, K),
                    dst_p.reshape(E_PAD // K, K)], axis=1)
    acc = scatter_kernel(sd, y)
    out = _out_call(acc, y, partials, b.reshape(1, CH))
    return out[:N_NODES]


# speedup vs baseline: 1.0200x; 1.0200x over previous
import functools

import jax
import jax.numpy as jnp
from jax import lax
from jax.experimental import pallas as pl
from jax.experimental.pallas import tpu as pltpu
from jax.experimental.pallas import tpu_sc as plsc

N_NODES = 10000
N_EDGES = 320000
CH = 128

NC, NS, L = 2, 16, 16
NW = NC * NS
N_PAD = 10240
E_PAD = 327680
EPT = E_PAD // NW
K = 128
N_CHUNKS = EPT // K
SPT = N_PAD // NS

NBUF = 2
D = 1
ISLOTS = 4
IL = 3
BODYN = 4

N0 = 132
N1 = 160 - N0
assert N0 % BODYN == 0 and N1 % BODYN == 0
assert NS * (N0 + N1) * K == E_PAD

ROW_BLK = 1024
N_BLKS = N_PAD // ROW_BLK


@functools.cache
def _sc_kernels():
    mesh = plsc.VectorSubcoreMesh(core_axis_name="c", subcore_axis_name="s",
                                  num_cores=NC, num_subcores=NS)


    @functools.partial(
        pl.kernel,
        out_type=jax.ShapeDtypeStruct((NW, N_PAD), jnp.float32),
        mesh=mesh,
        scratch_types=[
            pltpu.VMEM((N_PAD,), jnp.float32),
            pltpu.VMEM((EPT,), jnp.int32),
        ],
        compiler_params=pltpu.CompilerParams(needs_layout_passes=False),
    )
    def deg_kernel(dst_hbm, out_hbm, hist_v, idx_v):
        cid = lax.axis_index("c")
        sid = lax.axis_index("s")
        wid = sid * NC + cid
        zeros = jnp.zeros((L,), jnp.float32)
        ones = jnp.ones((L,), jnp.float32)

        def zero_body(i, carry):
            hist_v[pl.ds(i * L, L)] = zeros
            return carry
        lax.fori_loop(0, N_PAD // L, zero_body, 0, unroll=8)

        pltpu.sync_copy(dst_hbm.at[pl.ds(wid * EPT, EPT)], idx_v)

        def hist_body(i, carry):
            idx = idx_v[pl.ds(i * L, L)]
            plsc.addupdate_scatter(hist_v, [idx], ones)
            return carry
        lax.fori_loop(0, EPT // L, hist_body, 0, unroll=8)

        pltpu.sync_copy(hist_v, out_hbm.at[wid])


    @functools.partial(
        pl.kernel,
        out_type=jax.ShapeDtypeStruct((NC, N_PAD, CH), jnp.float32),
        mesh=mesh,
        scratch_types=[
            pltpu.VMEM((ISLOTS, 2, K), jnp.int32),
            pltpu.VMEM((NBUF, K, CH), jnp.float32),
            pltpu.VMEM_SHARED((N_PAD, CH), jnp.float32),
            pltpu.SemaphoreType.DMA((ISLOTS,)),
            pltpu.SemaphoreType.DMA((NBUF,)),
            pltpu.SemaphoreType.DMA((NBUF,)),
        ],
    )
    def scatter_kernel(sd_hbm, y_hbm, out_hbm,
                       sdv, rows, acc_sh, isem, gsem, ssem):
        cid = lax.axis_index("c")
        sid = lax.axis_index("s")
        cbase = jnp.where(cid == 0, sid * N0, NS * N0 + sid * N1)
        nch = jnp.where(cid == 0, N0, N1)

        def idx_copy(c, s):
            return pltpu.make_async_copy(
                sd_hbm.at[cbase + c], sdv.at[s], isem.at[s])

        for c in range(IL):
            idx_copy(c, c % ISLOTS).start()

        zeros = jnp.zeros((L,), jnp.float32)

        def zb(i, carry):
            rows[0, i // (CH // L), pl.ds((i % (CH // L)) * L, L)] = zeros
            return carry
        lax.fori_loop(0, K * (CH // L), zb, 0, unroll=8)

        def zcopy(j, carry):
            pltpu.sync_copy(rows.at[0],
                            acc_sh.at[pl.ds(sid * SPT + j * K, K)])
            return carry
        lax.fori_loop(0, SPT // K, zcopy, 0)

        plsc.subcore_barrier()

        def gather_copy(r, s):
            return pltpu.make_async_copy(
                y_hbm.at[sdv.at[s, 0]], rows.at[r], gsem.at[r])

        def scatter_copy(r, s):
            return pltpu.make_async_copy(
                rows.at[r], acc_sh.at[sdv.at[s, 1]], ssem.at[r])

        for c in range(D):
            idx_copy(c, c % ISLOTS).wait()
            gather_copy(c % NBUF, c % ISLOTS).start()

        def step(c, j):
            r, s = j % NBUF, j % ISLOTS
            gather_copy(r, s).wait()
            scatter_copy(r, s).start(add=True)

            @pl.when(c - (NBUF - D) >= 0)
            def _():
                scatter_copy((j + D) % NBUF, (j + D) % ISLOTS).wait()

            @pl.when(c + D < nch)
            def _():
                idx_copy(c + D, (j + D) % ISLOTS).wait()
                gather_copy((j + D) % NBUF, (j + D) % ISLOTS).start()

            @pl.when(c + IL < nch)
            def _():
                idx_copy(c + IL, (j + IL) % ISLOTS).start()

        def block(t, carry):
            for j in range(BODYN):
                step(t * BODYN + j, j)
            return carry
        lax.fori_loop(0, nch // BODYN, block, 0)

        for i in range(NBUF - D):
            c = BODYN - (NBUF - D) + i
            scatter_copy(c % NBUF, c % ISLOTS).wait()

        plsc.subcore_barrier()

        pltpu.sync_copy(acc_sh.at[pl.ds(sid * SPT, SPT)],
                        out_hbm.at[cid].at[pl.ds(sid * SPT, SPT)])

    return deg_kernel, scatter_kernel


def _y_body(x_ref, w_ref, p_ref, y_ref):
    deg = 1.0 + jnp.sum(p_ref[...], axis=0)
    dinv = lax.rsqrt(deg)
    xw = jnp.dot(x_ref[...], w_ref[...], preferred_element_type=jnp.float32)
    y_ref[...] = xw * dinv[:, None]


_y_call = pl.pallas_call(
    _y_body,
    out_shape=jax.ShapeDtypeStruct((N_PAD, CH), jnp.float32),
    grid=(N_BLKS,),
    in_specs=[
        pl.BlockSpec((ROW_BLK, CH), lambda i: (i, 0)),
        pl.BlockSpec((CH, CH), lambda i: (0, 0)),
        pl.BlockSpec((NW, ROW_BLK), lambda i: (0, i)),
    ],
    out_specs=pl.BlockSpec((ROW_BLK, CH), lambda i: (i, 0)),
)


def _out_body(a_ref, y_ref, p_ref, b_ref, o_ref):
    deg = 1.0 + jnp.sum(p_ref[...], axis=0)
    dinv = lax.rsqrt(deg)
    s = a_ref[0] + a_ref[1] + y_ref[...]
    o_ref[...] = jnp.maximum(s * dinv[:, None] + b_ref[...], 0.0)


_out_call = pl.pallas_call(
    _out_body,
    out_shape=jax.ShapeDtypeStruct((N_PAD, CH), jnp.float32),
    grid=(N_BLKS,),
    in_specs=[
        pl.BlockSpec((NC, ROW_BLK, CH), lambda i: (0, i, 0)),
        pl.BlockSpec((ROW_BLK, CH), lambda i: (i, 0)),
        pl.BlockSpec((NW, ROW_BLK), lambda i: (0, i)),
        pl.BlockSpec((1, CH), lambda i: (0, 0)),
    ],
    out_specs=pl.BlockSpec((ROW_BLK, CH), lambda i: (i, 0)),
)


def kernel(x, edge_index, W, b):
    deg_kernel, scatter_kernel = _sc_kernels()
    src = edge_index[0].astype(jnp.int32)
    dst = edge_index[1].astype(jnp.int32)
    pad = jnp.full((E_PAD - N_EDGES,), N_PAD - 1, jnp.int32)
    src_p = jnp.concatenate([src, pad])
    dst_p = jnp.concatenate([dst, pad])
    x_p = jnp.pad(x, ((0, N_PAD - N_NODES), (0, 0)))

    partials = deg_kernel(dst_p)
    y = _y_call(x_p, W, partials)
    sd = jnp.stack([src_p.reshape(E_PAD // K, K),
                    dst_p.reshape(E_PAD // K, K)], axis=1)
    acc = scatter_kernel(sd, y)
    out = _out_call(acc, y, partials, b.reshape(1, CH))
    return out[:N_NODES]
